# baseline (speedup 1.0000x reference)
import jax
import jax.numpy as jnp
from jax import lax
from jax.experimental import pallas as pl
from jax.experimental.pallas import tpu as pltpu

_EPS = 1e-5
_LANES = 128
_F = 16


def _body(idx_ref, *refs):
    n = pl.program_id(0)
    w_refs = refs[:_F]
    v_ref, gb_ref, out_ref = refs[_F:]
    dim = w_refs[0].shape[0]
    n_fields = idx_ref.shape[0]
    lane = lax.broadcasted_iota(jnp.int32, (dim, _LANES), 1)
    g = gb_ref[:, :1]
    b = gb_ref[:, 1:2]
    for k in range(_F):
        blk = w_refs[k][...]
        f = jnp.minimum(n * _F + k, n_fields - 1)
        rm = lax.rem(idx_ref[f], _LANES)
        col = jnp.sum(jnp.where(lane == rm, blk, 0.0),
                      axis=1, keepdims=True)
        mean = jnp.sum(col, axis=0, keepdims=True) / dim
        cent = col - mean
        var = jnp.sum(cent * cent, axis=0, keepdims=True) / dim
        c = cent * g
        v = v_ref[k:k + 1, :]
        s = v * lax.rsqrt(v * v * var + _EPS)
        out_ref[k] = s * c + b


def kernel(values, indices, W, ln_gamma, ln_beta):
    batch, n_fields = values.shape
    dim = W.shape[1]
    idx = indices.astype(jnp.int32)
    w_t = W.T
    v_t = values.T
    gb = jnp.stack([ln_gamma, ln_beta], axis=1)

    grid = (n_fields + _F - 1) // _F
    w_specs = [
        pl.BlockSpec(
            (dim, _LANES),
            lambda n, ix, k=k:
                (0, ix[jnp.minimum(n * _F + k, n_fields - 1)] // _LANES))
        for k in range(_F)
    ]
    out3 = pl.pallas_call(
        _body,
        grid_spec=pltpu.PrefetchScalarGridSpec(
            num_scalar_prefetch=1,
            grid=(grid,),
            in_specs=w_specs + [
                pl.BlockSpec((_F, batch), lambda n, ix: (n, 0)),
                pl.BlockSpec((dim, 2), lambda n, ix: (0, 0)),
            ],
            out_specs=pl.BlockSpec(
                (_F, dim, batch), lambda n, ix: (n, 0, 0)),
        ),
        out_shape=jax.ShapeDtypeStruct((n_fields, dim, batch), jnp.float32),
        compiler_params=pltpu.CompilerParams(
            dimension_semantics=("parallel",)),
    )(idx, *([w_t] * _F), v_t, gb)
    return out3.transpose(2, 0, 1)


# speedup vs baseline: 1.1353x; 1.1353x over previous
import jax
import jax.numpy as jnp
from jax import lax
from jax.experimental import pallas as pl
from jax.experimental.pallas import tpu as pltpu

_EPS = 1e-5
_LANES = 128
_F = 8


def _body(idx_ref, *refs):
    n = pl.program_id(0)
    w_refs = refs[:_F]
    v_ref, g_ref, b_ref, out_ref = refs[_F:]
    dim = w_refs[0].shape[0]
    n_fields = idx_ref.shape[0]
    lane = lax.broadcasted_iota(jnp.int32, (dim, _LANES), 1)
    g = g_ref[...].reshape(dim, 1)
    b = b_ref[...].reshape(dim, 1)
    for k in range(_F):
        blk = w_refs[k][...]
        f = jnp.minimum(n * _F + k, n_fields - 1)
        rm = lax.rem(idx_ref[f], _LANES)
        col = jnp.sum(jnp.where(lane == rm, blk, 0.0),
                      axis=1, keepdims=True)
        mean = jnp.sum(col, axis=0, keepdims=True) / dim
        cent = col - mean
        var = jnp.sum(cent * cent, axis=0, keepdims=True) / dim
        c = cent * g
        v = v_ref[k:k + 1, :]
        s = v * lax.rsqrt(v * v * var + _EPS)
        out_ref[k] = s * c + b


def kernel(values, indices, W, ln_gamma, ln_beta):
    batch, n_fields = values.shape
    dim = W.shape[1]
    idx = indices.astype(jnp.int32)
    w_t = W.T
    v_t = values.T

    grid = (n_fields + _F - 1) // _F
    w_specs = [
        pl.BlockSpec(
            (dim, _LANES),
            lambda n, ix, k=k:
                (0, ix[jnp.minimum(n * _F + k, n_fields - 1)] // _LANES))
        for k in range(_F)
    ]
    out3 = pl.pallas_call(
        _body,
        grid_spec=pltpu.PrefetchScalarGridSpec(
            num_scalar_prefetch=1,
            grid=(grid,),
            in_specs=w_specs + [
                pl.BlockSpec((_F, batch), lambda n, ix: (n, 0)),
                pl.BlockSpec((dim,), lambda n, ix: (0,)),
                pl.BlockSpec((dim,), lambda n, ix: (0,)),
            ],
            out_specs=pl.BlockSpec(
                (_F, dim, batch), lambda n, ix: (n, 0, 0)),
        ),
        out_shape=jax.ShapeDtypeStruct((n_fields, dim, batch), jnp.float32),
        compiler_params=pltpu.CompilerParams(
            dimension_semantics=("parallel",)),
    )(idx, *([w_t] * _F), v_t, ln_gamma, ln_beta)
    return out3.transpose(2, 0, 1)
